# Initial kernel scaffold; baseline (speedup 1.0000x reference)
import jax
import jax.numpy as jnp
from jax.experimental import pallas as pl


def kernel(x_0, x_1, x_2, neighborhood_0_to_0, neighborhood_1_to_1, neighborhood_2_to_2, neighborhood_0_to_1, neighborhood_1_to_2, hbs0_l1_W, hbs0_l1_a, hbns01_l1_ws, hbns01_l1_wt, hbns01_l1_a, hbns12_l1_ws, hbns12_l1_wt, hbns12_l1_a, hbs0_l2_W, hbs0_l2_a, hbns01_l2_ws, hbns01_l2_wt, hbns01_l2_a, hbs1_l2_W, hbs1_l2_a, hbns12_l2_ws, hbns12_l2_wt, hbns12_l2_a, hbs2_l2_W, hbs2_l2_a):
    raise NotImplementedError("write your pallas kernel here")


# speedup vs baseline: 1.0572x; 1.0572x over previous
import functools

import jax
import jax.numpy as jnp
from jax.experimental import pallas as pl
from jax.experimental.pallas import tpu as pltpu

_NEG_SLOPE = 0.2
_BI = 256


def _lrelu(x):
    return jnp.where(x >= 0, x, _NEG_SLOPE * x)


def _dot(a, b, dims):
    return jax.lax.dot_general(a, b, (dims, ((), ())),
                               preferred_element_type=jnp.float32)


def _row_attn_kernel(xs_ref, xt_ref, ws_ref, wt_ref, att_ref, a_ref,
                     o_ref, sm_buf, tm_buf, *, bi, rows_first):
    i = pl.program_id(0)

    @pl.when(i == 0)
    def _():
        sm_buf[...] = _dot(xs_ref[...], ws_ref[...], (((1,), (0,))))
        tm_buf[...] = _dot(xt_ref[...], wt_ref[...], (((1,), (0,))))

    ar = att_ref[0:1, :] if rows_first else att_ref[1:2, :]
    ac = att_ref[1:2, :] if rows_first else att_ref[0:1, :]
    sm = sm_buf[...]
    tm_i = tm_buf[pl.ds(i * bi, bi), :]
    u = _dot(tm_i, ar, (((1,), (1,))))
    v = _dot(ac, sm, (((1,), (1,))))
    mask = (a_ref[...] != 0).astype(jnp.float32)
    p = jnp.exp(_lrelu(u + v) - _lrelu(u + jnp.max(v))) * mask
    den = jnp.maximum(jnp.sum(p, axis=1, keepdims=True), 1e-30)
    num = _dot(p, sm, (((1,), (0,))))
    o_ref[...] = jnp.maximum(num / den, 0.0)


def _row_attn(xs, xt, ws, wt, att2, A, rows_first):
    n_t, n_s = A.shape
    d = ws.shape[1]
    bi = min(_BI, n_t)
    return pl.pallas_call(
        functools.partial(_row_attn_kernel, bi=bi, rows_first=rows_first),
        grid=(n_t // bi,),
        in_specs=[
            pl.BlockSpec(xs.shape, lambda i: (0, 0)),
            pl.BlockSpec(xt.shape, lambda i: (0, 0)),
            pl.BlockSpec(ws.shape, lambda i: (0, 0)),
            pl.BlockSpec(wt.shape, lambda i: (0, 0)),
            pl.BlockSpec((2, d), lambda i: (0, 0)),
            pl.BlockSpec((bi, n_s), lambda i: (i, 0)),
        ],
        out_specs=pl.BlockSpec((bi, d), lambda i: (i, 0)),
        out_shape=jax.ShapeDtypeStruct((n_t, d), jnp.float32),
        scratch_shapes=[pltpu.VMEM((n_s, d), jnp.float32),
                        pltpu.VMEM((n_t, d), jnp.float32)],
    )(xs, xt, ws, wt, att2, A)


def _hbns_kernel(xs_ref, xt_ref, ws_ref, wt_ref, att_ref, a_ref,
                 oe_ref, of_ref, sm_buf, tm_buf, fnum, fden, ones_buf,
                 *, bi, nsteps):
    i = pl.program_id(0)

    @pl.when(i == 0)
    def _():
        sm_buf[...] = _dot(xs_ref[...], ws_ref[...], (((1,), (0,))))
        tm_buf[...] = _dot(xt_ref[...], wt_ref[...], (((1,), (0,))))
        fnum[...] = jnp.zeros_like(fnum)
        fden[...] = jnp.zeros_like(fden)
        ones_buf[...] = jnp.ones_like(ones_buf)

    a0 = att_ref[0:1, :]
    a1 = att_ref[1:2, :]
    sm = sm_buf[...]
    tm_i = tm_buf[pl.ds(i * bi, bi), :]
    p = _dot(a0, sm, (((1,), (1,))))
    q = _dot(a1, sm, (((1,), (1,))))
    r_all = _dot(a0, tm_buf[...], (((1,), (1,))))
    s_i = _dot(tm_i, a1, (((1,), (1,))))
    r_i = _dot(tm_i, a0, (((1,), (1,))))
    mask = (a_ref[...] != 0).astype(jnp.float32)

    pe = jnp.exp(_lrelu(s_i + p) - _lrelu(s_i + jnp.max(p))) * mask
    den_e = jnp.maximum(jnp.sum(pe, axis=1, keepdims=True), 1e-30)
    oe_ref[...] = jnp.maximum(_dot(pe, sm, (((1,), (0,)))) / den_e, 0.0)

    pf = jnp.exp(_lrelu(r_i + q) - _lrelu(q + jnp.max(r_all))) * mask
    fnum[...] += _dot(pf, tm_i, (((0,), (0,))))
    fden[...] += _dot(pf, ones_buf[...], (((0,), (0,))))

    @pl.when(i == nsteps - 1)
    def _():
        of_ref[...] = jnp.maximum(
            fnum[...] / jnp.maximum(fden[...], 1e-30), 0.0)


def _hbns(xs, xt, ws, wt, att2, A):
    n_t, n_s = A.shape
    d = ws.shape[1]
    bi = min(_BI, n_t)
    nsteps = n_t // bi
    oe, of = pl.pallas_call(
        functools.partial(_hbns_kernel, bi=bi, nsteps=nsteps),
        grid=(nsteps,),
        in_specs=[
            pl.BlockSpec(xs.shape, lambda i: (0, 0)),
            pl.BlockSpec(xt.shape, lambda i: (0, 0)),
            pl.BlockSpec(ws.shape, lambda i: (0, 0)),
            pl.BlockSpec(wt.shape, lambda i: (0, 0)),
            pl.BlockSpec((2, d), lambda i: (0, 0)),
            pl.BlockSpec((bi, n_s), lambda i: (i, 0)),
        ],
        out_specs=[
            pl.BlockSpec((bi, d), lambda i: (i, 0)),
            pl.BlockSpec((n_s, d), lambda i: (0, 0)),
        ],
        out_shape=[jax.ShapeDtypeStruct((n_t, d), jnp.float32),
                   jax.ShapeDtypeStruct((n_s, d), jnp.float32)],
        scratch_shapes=[pltpu.VMEM((n_s, d), jnp.float32),
                        pltpu.VMEM((n_t, d), jnp.float32),
                        pltpu.VMEM((n_s, d), jnp.float32),
                        pltpu.VMEM((n_s, 1), jnp.float32),
                        pltpu.VMEM((bi, 1), jnp.float32)],
    )(xs, xt, ws, wt, att2, A)
    return of, oe


def kernel(x_0, x_1, x_2, neighborhood_0_to_0, neighborhood_1_to_1,
           neighborhood_2_to_2, neighborhood_0_to_1, neighborhood_1_to_2,
           hbs0_l1_W, hbs0_l1_a, hbns01_l1_ws, hbns01_l1_wt, hbns01_l1_a,
           hbns12_l1_ws, hbns12_l1_wt, hbns12_l1_a,
           hbs0_l2_W, hbs0_l2_a, hbns01_l2_ws, hbns01_l2_wt, hbns01_l2_a,
           hbs1_l2_W, hbs1_l2_a, hbns12_l2_ws, hbns12_l2_wt, hbns12_l2_a,
           hbs2_l2_W, hbs2_l2_a):
    def hbs(x, A, W, att):
        return _row_attn(x, x, W, W, att.reshape(2, -1), A, rows_first=True)

    def hbns(xs, xt, A, ws, wt, att):
        return _hbns(xs, xt, ws, wt, att.reshape(2, -1), A)

    def hbns_e_only(xs, xt, A, ws, wt, att):
        return _row_attn(xs, xt, ws, wt, att.reshape(2, -1), A,
                         rows_first=False)

    x_0_to_0 = hbs(x_0, neighborhood_0_to_0, hbs0_l1_W, hbs0_l1_a)
    x_0_to_1, x_1_to_0 = hbns(x_1, x_0, neighborhood_0_to_1,
                              hbns01_l1_ws, hbns01_l1_wt, hbns01_l1_a)
    x_1_to_2, x_2_to_1 = hbns(x_2, x_1, neighborhood_1_to_2,
                              hbns12_l1_ws, hbns12_l1_wt, hbns12_l1_a)
    x_0_l1 = jax.nn.relu(x_0_to_0 + x_1_to_0)
    x_1_l1 = jax.nn.relu(x_0_to_1 + x_2_to_1)
    x_2_l1 = jax.nn.relu(x_1_to_2)
    x_0_to_0 = hbs(x_0_l1, neighborhood_0_to_0, hbs0_l2_W, hbs0_l2_a)
    x_0_to_1, x_1_to_0 = hbns(x_1_l1, x_0_l1, neighborhood_0_to_1,
                              hbns01_l2_ws, hbns01_l2_wt, hbns01_l2_a)
    x_1_to_1 = hbs(x_1_l1, neighborhood_1_to_1, hbs1_l2_W, hbs1_l2_a)
    x_2_to_1 = hbns_e_only(x_2_l1, x_1_l1, neighborhood_1_to_2,
                           hbns12_l2_ws, hbns12_l2_wt, hbns12_l2_a)
    x_0_out = jax.nn.relu(x_0_to_0 + x_1_to_0)
    x_1_out = jax.nn.relu(x_0_to_1 + x_1_to_1 + x_2_to_1)
    return (x_0_out, x_1_out)
